# Initial kernel scaffold; baseline (speedup 1.0000x reference)
import jax
import jax.numpy as jnp
from jax.experimental import pallas as pl


def kernel(x, edge_index, edge_attr, params):
    raise NotImplementedError("write your pallas kernel here")


# speedup vs baseline: 4.3014x; 4.3014x over previous
import functools

import jax
import jax.numpy as jnp
from jax import lax
from jax.experimental import pallas as pl
from jax.experimental.pallas import tpu as pltpu
from jax.experimental.pallas import tpu_sc as plsc

N = 10000
E = 320000
F_IN = 128
H = 16

NC = 2
NS = 16
NW = NC * NS
EW = E // NW
C = 80
NCH = EW // C
ZR = N // NS

_mesh = lambda: plsc.VectorSubcoreMesh(core_axis_name="c", subcore_axis_name="s")
_sc_params = lambda: pltpu.CompilerParams(use_tc_tiling_on_sc=False)


def _sc_gather2(tab_a, idx_a, tab_b, idx_b):
    @functools.partial(
        pl.kernel,
        out_type=(jax.ShapeDtypeStruct((E, H), jnp.float32),
                  jax.ShapeDtypeStruct((E, H), jnp.float32)),
        mesh=_mesh(),
        compiler_params=_sc_params(),
        scratch_types=[
            pltpu.VMEM((NCH, C), jnp.int32), pltpu.VMEM((NCH, C), jnp.int32),
            pltpu.VMEM((C, H), jnp.float32), pltpu.VMEM((C, H), jnp.float32),
            pltpu.VMEM((C, H), jnp.float32), pltpu.VMEM((C, H), jnp.float32),
            pltpu.SemaphoreType.DMA, pltpu.SemaphoreType.DMA,
            pltpu.SemaphoreType.DMA, pltpu.SemaphoreType.DMA,
        ],
    )
    def k(ta, ia, tb, ib, oa, ob, iva, ivb, ba0, ba1, bb0, bb1,
          sa0, sa1, sb0, sb1):
        w = lax.axis_index("s") * NC + lax.axis_index("c")
        base = w * EW
        pltpu.sync_copy(ia.at[w], iva)
        pltpu.sync_copy(ib.at[w], ivb)

        def issue(j, buf_a, buf_b, sem_a, sem_b):
            ca = pltpu.async_copy(ta.at[iva.at[j]], buf_a, sem_a)
            cb = pltpu.async_copy(tb.at[ivb.at[j]], buf_b, sem_b)
            return ca, cb

        ca0, cb0 = issue(0, ba0, bb0, sa0, sb0)

        def body(t, _):
            j0 = 2 * t
            j1 = j0 + 1
            ca1, cb1 = issue(j1, ba1, bb1, sa1, sb1)
            pltpu.make_async_copy(ta.at[iva.at[j0]], ba0, sa0).wait()
            pltpu.make_async_copy(tb.at[ivb.at[j0]], bb0, sb0).wait()
            pltpu.sync_copy(ba0, oa.at[pl.ds(base + j0 * C, C)])
            pltpu.sync_copy(bb0, ob.at[pl.ds(base + j0 * C, C)])
            issue(j0 + 2, ba0, bb0, sa0, sb0)
            ca1.wait()
            cb1.wait()
            pltpu.sync_copy(ba1, oa.at[pl.ds(base + j1 * C, C)])
            pltpu.sync_copy(bb1, ob.at[pl.ds(base + j1 * C, C)])
            return 0

        lax.fori_loop(0, (NCH - 1) // 2, body, 0)
        jt = NCH - 1
        pltpu.make_async_copy(ta.at[iva.at[jt]], ba0, sa0).wait()
        pltpu.make_async_copy(tb.at[ivb.at[jt]], bb0, sb0).wait()
        pltpu.sync_copy(ba0, oa.at[pl.ds(base + jt * C, C)])
        pltpu.sync_copy(bb0, ob.at[pl.ds(base + jt * C, C)])

    return k(tab_a, idx_a, tab_b, idx_b)


def _sc_scatter2(vals_a, idx_a, vals_b, idx_b):
    @functools.partial(
        pl.kernel,
        out_type=(jax.ShapeDtypeStruct((NC, N, H), jnp.float32),
                  jax.ShapeDtypeStruct((NC, N, H), jnp.float32)),
        mesh=_mesh(),
        compiler_params=_sc_params(),
        scratch_types=[
            pltpu.VMEM((NCH, C), jnp.int32), pltpu.VMEM((NCH, C), jnp.int32),
            pltpu.VMEM((C, H), jnp.float32), pltpu.VMEM((C, H), jnp.float32),
            pltpu.VMEM((C, H), jnp.float32), pltpu.VMEM((C, H), jnp.float32),
            pltpu.VMEM((ZR, H), jnp.float32),
            pltpu.VMEM_SHARED((N, H), jnp.float32),
            pltpu.VMEM_SHARED((N, H), jnp.float32),
            pltpu.SemaphoreType.DMA, pltpu.SemaphoreType.DMA,
            pltpu.SemaphoreType.DMA, pltpu.SemaphoreType.DMA,
        ],
    )
    def k(va, ia, vb, ib, oa, ob, iva, ivb, ba0, ba1, bb0, bb1, zbuf,
          acc_a, acc_b, sa0, sa1, sb0, sb1):
        c = lax.axis_index("c")
        s = lax.axis_index("s")
        w = s * NC + c

        def zrow(i, _):
            zbuf[i, :] = jnp.zeros((H,), jnp.float32)
            return 0
        lax.fori_loop(0, ZR, zrow, 0)
        pltpu.sync_copy(zbuf, acc_a.at[pl.ds(s * ZR, ZR)])
        pltpu.sync_copy(zbuf, acc_b.at[pl.ds(s * ZR, ZR)])

        pltpu.sync_copy(ia.at[w], iva)
        pltpu.sync_copy(ib.at[w], ivb)
        plsc.subcore_barrier()

        def issue(j, buf_a, buf_b, sem_a, sem_b):
            ca = pltpu.async_copy(va.at[w, j], buf_a, sem_a)
            cb = pltpu.async_copy(vb.at[w, j], buf_b, sem_b)
            return ca, cb

        issue(0, ba0, bb0, sa0, sb0)

        def body(t, _):
            j0 = 2 * t
            j1 = j0 + 1
            issue(j1, ba1, bb1, sa1, sb1)
            pltpu.make_async_copy(va.at[w, j0], ba0, sa0).wait()
            pltpu.sync_copy(ba0, acc_a.at[iva.at[j0]], add=True)
            pltpu.make_async_copy(vb.at[w, j0], bb0, sb0).wait()
            pltpu.sync_copy(bb0, acc_b.at[ivb.at[j0]], add=True)
            issue(j0 + 2, ba0, bb0, sa0, sb0)
            pltpu.make_async_copy(va.at[w, j1], ba1, sa1).wait()
            pltpu.sync_copy(ba1, acc_a.at[iva.at[j1]], add=True)
            pltpu.make_async_copy(vb.at[w, j1], bb1, sb1).wait()
            pltpu.sync_copy(bb1, acc_b.at[ivb.at[j1]], add=True)
            return 0

        lax.fori_loop(0, (NCH - 1) // 2, body, 0)
        jt = NCH - 1
        pltpu.make_async_copy(va.at[w, jt], ba0, sa0).wait()
        pltpu.sync_copy(ba0, acc_a.at[iva.at[jt]], add=True)
        pltpu.make_async_copy(vb.at[w, jt], bb0, sb0).wait()
        pltpu.sync_copy(bb0, acc_b.at[ivb.at[jt]], add=True)

        plsc.subcore_barrier()
        pltpu.sync_copy(acc_a.at[pl.ds(s * ZR, ZR)],
                        oa.at[c, pl.ds(s * ZR, ZR)])
        pltpu.sync_copy(acc_b.at[pl.ds(s * ZR, ZR)],
                        ob.at[c, pl.ds(s * ZR, ZR)])

    return k(vals_a, idx_a, vals_b, idx_b)


def _tc_node_emb(x, W, b):
    def body(x_r, W_r, b_r, o_r):
        o_r[...] = jnp.dot(x_r[...], W_r[...],
                           preferred_element_type=jnp.float32) + b_r[...]

    return pl.pallas_call(
        body,
        out_shape=jax.ShapeDtypeStruct((N, H), jnp.float32),
    )(x, W, b.reshape(1, H))


_BE = 2000


def _tc_edge_emb(ea, W, b):
    def body(ea_r, W_r, b_r, o_r):
        o_r[...] = jnp.dot(ea_r[...], W_r[...],
                           preferred_element_type=jnp.float32) + b_r[...]

    g = E // _BE
    return pl.pallas_call(
        body,
        grid=(g,),
        in_specs=[pl.BlockSpec((_BE, 16), lambda i: (i, 0)),
                  pl.BlockSpec((16, H), lambda i: (0, 0)),
                  pl.BlockSpec((1, H), lambda i: (0, 0))],
        out_specs=pl.BlockSpec((_BE, H), lambda i: (i, 0)),
        out_shape=jax.ShapeDtypeStruct((E, H), jnp.float32),
    )(ea, W, b.reshape(1, H))


def _tc_edge_alpha(hs, hd, e, Wsi, Wdi, Wei, bi, Wso, Wdo, Weo, bo):
    def body(hs_r, hd_r, e_r, wsi, wdi, wei, bi_r, wso, wdo, weo, bo_r, o_r):
        hsv, hdv, ev = hs_r[...], hd_r[...], e_r[...]
        ai = (jnp.dot(hsv, wsi[...], preferred_element_type=jnp.float32)
              + jnp.dot(hdv, wdi[...], preferred_element_type=jnp.float32)
              + jnp.dot(ev, wei[...], preferred_element_type=jnp.float32)
              + bi_r[...])
        ao = (jnp.dot(hsv, wso[...], preferred_element_type=jnp.float32)
              + jnp.dot(hdv, wdo[...], preferred_element_type=jnp.float32)
              + jnp.dot(ev, weo[...], preferred_element_type=jnp.float32)
              + bo_r[...])
        ai = jnp.where(ai > 0, ai, 0.2 * ai)
        ao = jnp.where(ao > 0, ao, 0.2 * ao)
        o_r[...] = jnp.concatenate(
            [jnp.exp(ai), jnp.exp(ao),
             jnp.zeros((_BE, 8), jnp.float32)], axis=1)

    g = E // _BE
    wspec = pl.BlockSpec((H, 4), lambda i: (0, 0))
    bspec = pl.BlockSpec((1, 4), lambda i: (0, 0))
    espec = pl.BlockSpec((_BE, H), lambda i: (i, 0))
    return pl.pallas_call(
        body,
        grid=(g,),
        in_specs=[espec, espec, espec,
                  wspec, wspec, wspec, bspec,
                  wspec, wspec, wspec, bspec],
        out_specs=espec,
        out_shape=jax.ShapeDtypeStruct((E, H), jnp.float32),
    )(hs, hd, e, Wsi, Wdi, Wei, bi.reshape(1, 4),
      Wso, Wdo, Weo, bo.reshape(1, 4))


def _tc_combine_S(Sa, Sb):
    def body(sa_r, sb_r, oa_r, ob_r):
        si = sa_r[0, :, 0:4] + sa_r[1, :, 0:4]
        so = sb_r[0, :, 4:8] + sb_r[1, :, 4:8]
        z = jnp.zeros((N, 12), jnp.float32)
        oa_r[...] = jnp.concatenate([1.0 / (si + 1e-16), z], axis=1)
        ob_r[...] = jnp.concatenate([1.0 / (so + 1e-16), z], axis=1)

    return pl.pallas_call(
        body,
        out_shape=(jax.ShapeDtypeStruct((N, H), jnp.float32),
                   jax.ShapeDtypeStruct((N, H), jnp.float32)),
    )(Sa, Sb)


def _tc_edge_msg(P, rga, rgb, hs, hd, e):
    def body(p_r, ra_r, rb_r, hs_r, hd_r, e_r, oi_r, oo_r):
        pv, ev = p_r[...], e_r[...]
        am_in = jnp.sum(pv[:, 0:4] * ra_r[:, 0:4], axis=1, keepdims=True) * 0.25
        am_out = jnp.sum(pv[:, 4:8] * rb_r[:, 0:4], axis=1, keepdims=True) * 0.25
        oi_r[...] = jnp.maximum(hs_r[...] + ev * am_in, 0.0)
        oo_r[...] = jnp.maximum(hd_r[...] + ev * am_out, 0.0)

    g = E // _BE
    espec = pl.BlockSpec((_BE, H), lambda i: (i, 0))
    return pl.pallas_call(
        body,
        grid=(g,),
        in_specs=[espec] * 6,
        out_specs=(espec, espec),
        out_shape=(jax.ShapeDtypeStruct((E, H), jnp.float32),
                   jax.ShapeDtypeStruct((E, H), jnp.float32)),
    )(P, rga, rgb, hs, hd, e)


def _tc_node_update(h, Aa, Ab, lp):
    ci, co = lp["conv_in"], lp["conv_out"]
    eps2 = jnp.stack([ci["eps"], co["eps"]]).reshape(1, 2)

    def bn_(z, g, b):
        mu = jnp.mean(z, axis=0, keepdims=True)
        var = jnp.mean((z - mu) ** 2, axis=0, keepdims=True)
        return (z - mu) * lax.rsqrt(var + 1e-5) * g + b

    def conv_(h, aggr, eps, w1, b1, g1, bb1, w2, b2, g2, bb2):
        z = (1.0 + eps) * h + aggr
        z = jnp.dot(z, w1[...], preferred_element_type=jnp.float32) + b1[...]
        z = jnp.maximum(bn_(z, g1[...], bb1[...]), 0.0)
        z = jnp.dot(z, w2[...], preferred_element_type=jnp.float32) + b2[...]
        return jnp.maximum(bn_(z, g2[...], bb2[...]), 0.0)

    def body(h_r, aa_r, ab_r, eps_r,
             w1i, b1i, g1i, bb1i, w2i, b2i, g2i, bb2i,
             w1o, b1o, g1o, bb1o, w2o, b2o, g2o, bb2o,
             gn, bb, o_r):
        hv = h_r[...]
        aggr_in = aa_r[0] + aa_r[1]
        aggr_out = ab_r[0] + ab_r[1]
        h_in = conv_(hv, aggr_in, eps_r[0, 0],
                     w1i, b1i, g1i, bb1i, w2i, b2i, g2i, bb2i)
        h_out = conv_(hv, aggr_out, eps_r[0, 1],
                      w1o, b1o, g1o, bb1o, w2o, b2o, g2o, bb2o)
        m = bn_((h_in + h_out) * 0.5, gn[...], bb[...])
        o_r[...] = (hv + m) * 0.5

    def args(cp):
        return (cp["lin1"]["W"], cp["lin1"]["b"].reshape(1, -1),
                cp["bn1"]["g"].reshape(1, -1), cp["bn1"]["b"].reshape(1, -1),
                cp["lin2"]["W"], cp["lin2"]["b"].reshape(1, -1),
                cp["bn2"]["g"].reshape(1, -1), cp["bn2"]["b"].reshape(1, -1))

    return pl.pallas_call(
        body,
        out_shape=jax.ShapeDtypeStruct((N, H), jnp.float32),
    )(h, Aa, Ab, eps2, *args(ci), *args(co),
      lp["bn"]["g"].reshape(1, H), lp["bn"]["b"].reshape(1, H))


def _tc_final(h, W, b):
    def body(h_r, w_r, b_r, o_r):
        o_r[...] = jnp.dot(h_r[...], w_r[...],
                           preferred_element_type=jnp.float32) + b_r[...]

    return pl.pallas_call(
        body,
        out_shape=jax.ShapeDtypeStruct((N, 1), jnp.float32),
    )(h, W, b.reshape(1, 1))


def kernel(x, edge_index, edge_attr, params):
    src = edge_index[0]
    dst = edge_index[1]
    srcW = src.reshape(NW, NCH, C)
    dstW = dst.reshape(NW, NCH, C)

    h = _tc_node_emb(x, params["node_emb"]["W"], params["node_emb"]["b"])
    e = _tc_edge_emb(edge_attr, params["edge_emb"]["W"], params["edge_emb"]["b"])

    for lp in params["layers"]:
        ai, ao = lp["att_in"], lp["att_out"]
        hs, hd = _sc_gather2(h, srcW, h, dstW)
        P = _tc_edge_alpha(hs, hd, e,
                           ai["Ws"], ai["Wd"], ai["We"], ai["bias"],
                           ao["Ws"], ao["Wd"], ao["We"], ao["bias"])
        Pw = P.reshape(NW, NCH, C, H)
        Sa, Sb = _sc_scatter2(Pw, dstW, Pw, srcW)
        rSa, rSb = _tc_combine_S(Sa, Sb)
        rga, rgb = _sc_gather2(rSa, dstW, rSb, srcW)
        mi, mo = _tc_edge_msg(P, rga, rgb, hs, hd, e)
        Aa, Ab = _sc_scatter2(mi.reshape(NW, NCH, C, H), dstW,
                              mo.reshape(NW, NCH, C, H), srcW)
        h = _tc_node_update(h, Aa, Ab, lp)

    return _tc_final(h, params["mlp_node"]["W"], params["mlp_node"]["b"])


# speedup vs baseline: 18.8978x; 4.3934x over previous
import functools

import jax
import jax.numpy as jnp
from jax import lax
from jax.experimental import pallas as pl
from jax.experimental.pallas import tpu as pltpu
from jax.experimental.pallas import tpu_sc as plsc

N = 10000
E = 320000
F_IN = 128
H = 16
E8 = E * H // 128

NC = 2
NS = 16
NW = NC * NS
EW = E // NW
C = 80
NCH = EW // C
NB = 4
NR = (NCH - 1) // NB
ZR = N // NS

_mesh = lambda: plsc.VectorSubcoreMesh(core_axis_name="c", subcore_axis_name="s")
_sc_params = lambda: pltpu.CompilerParams(use_tc_tiling_on_sc=False)


def _sc_gather2(tab_a, idx_a, tab_b, idx_b):
    scratch = [pltpu.VMEM((NCH, C), jnp.int32), pltpu.VMEM((NCH, C), jnp.int32)]
    scratch += [pltpu.VMEM((C, H), jnp.float32)] * (2 * NB)
    scratch += [pltpu.SemaphoreType.DMA] * (4 * NB)

    @functools.partial(
        pl.kernel,
        out_type=(jax.ShapeDtypeStruct((E, H), jnp.float32),
                  jax.ShapeDtypeStruct((E, H), jnp.float32)),
        mesh=_mesh(),
        compiler_params=_sc_params(),
        scratch_types=scratch,
    )
    def k(ta, ia, tb, ib, oa, ob, iva, ivb, *bufsem):
        bum = bufsem[:2 * NB]
        sems = bufsem[2 * NB:]
        ba = bum[:NB]
        bb = bum[NB:]
        ga = sems[:NB]
        gb = sems[NB:2 * NB]
        ta_s = sems[2 * NB:3 * NB]
        tb_s = sems[3 * NB:]
        w = lax.axis_index("s") * NC + lax.axis_index("c")
        base = w * EW
        pltpu.sync_copy(ia.at[w], iva)
        pltpu.sync_copy(ib.at[w], ivb)

        def g_issue(j, i):
            pltpu.async_copy(ta.at[iva.at[j]], ba[i], ga[i])
            pltpu.async_copy(tb.at[ivb.at[j]], bb[i], gb[i])

        def g_wait(j, i):
            pltpu.make_async_copy(ta.at[iva.at[j]], ba[i], ga[i]).wait()
            pltpu.make_async_copy(tb.at[ivb.at[j]], bb[i], gb[i]).wait()

        def st_issue(j, i):
            pltpu.async_copy(ba[i], oa.at[pl.ds(base + j * C, C)], ta_s[i])
            pltpu.async_copy(bb[i], ob.at[pl.ds(base + j * C, C)], tb_s[i])

        def st_wait(j, i):
            pltpu.make_async_copy(ba[i], oa.at[pl.ds(base + j * C, C)],
                                  ta_s[i]).wait()
            pltpu.make_async_copy(bb[i], ob.at[pl.ds(base + j * C, C)],
                                  tb_s[i]).wait()

        for i in range(NB):
            g_issue(i, i)

        def body(t, _):
            j = NB * t
            for i in range(NB):
                g_wait(j + i, i)
                st_issue(j + i, i)
            for i in range(NB):
                st_wait(j + i, i)
                nj = j + NB + i

                @pl.when(nj < NCH)
                def _():
                    g_issue(nj, i)
            return 0

        lax.fori_loop(0, NR, body, 0)
        for i in range(NCH - NR * NB):
            jt = NR * NB + i
            g_wait(jt, i)
            st_issue(jt, i)
        for i in range(NCH - NR * NB):
            jt = NR * NB + i
            st_wait(jt, i)

    return k(tab_a, idx_a, tab_b, idx_b)


def _sc_scatter2(vals_a, idx_a, vals_b, idx_b):
    scratch = [pltpu.VMEM((NCH, C), jnp.int32), pltpu.VMEM((NCH, C), jnp.int32)]
    scratch += [pltpu.VMEM((C, H), jnp.float32)] * (2 * NB)
    scratch += [pltpu.VMEM((ZR, H), jnp.float32),
                pltpu.VMEM_SHARED((N, H), jnp.float32),
                pltpu.VMEM_SHARED((N, H), jnp.float32)]
    scratch += [pltpu.SemaphoreType.DMA] * (2 * NB)

    @functools.partial(
        pl.kernel,
        out_type=(jax.ShapeDtypeStruct((NC, N, H), jnp.float32),
                  jax.ShapeDtypeStruct((NC, N, H), jnp.float32)),
        mesh=_mesh(),
        compiler_params=_sc_params(),
        scratch_types=scratch,
    )
    def k(va, ia, vb, ib, oa, ob, iva, ivb, *rest):
        ba = rest[:NB]
        bb = rest[NB:2 * NB]
        zbuf, acc_a, acc_b = rest[2 * NB:2 * NB + 3]
        sems = rest[2 * NB + 3:]
        la = sems[:NB]
        lb = sems[NB:]
        c = lax.axis_index("c")
        s = lax.axis_index("s")
        w = s * NC + c

        def zrow(i, _):
            zbuf[i, :] = jnp.zeros((H,), jnp.float32)
            return 0
        lax.fori_loop(0, ZR, zrow, 0)
        pltpu.sync_copy(zbuf, acc_a.at[pl.ds(s * ZR, ZR)])
        pltpu.sync_copy(zbuf, acc_b.at[pl.ds(s * ZR, ZR)])

        pltpu.sync_copy(ia.at[w], iva)
        pltpu.sync_copy(ib.at[w], ivb)
        plsc.subcore_barrier()

        def l_issue(j, i):
            pltpu.async_copy(va.at[w, j], ba[i], la[i])
            pltpu.async_copy(vb.at[w, j], bb[i], lb[i])

        def l_wait(j, i):
            pltpu.make_async_copy(va.at[w, j], ba[i], la[i]).wait()
            pltpu.make_async_copy(vb.at[w, j], bb[i], lb[i]).wait()

        def scat(j, i):
            pltpu.sync_copy(ba[i], acc_a.at[iva.at[j]], add=True)
            pltpu.sync_copy(bb[i], acc_b.at[ivb.at[j]], add=True)

        for i in range(NB):
            l_issue(i, i)

        def body(t, _):
            j = NB * t
            for i in range(NB):
                l_wait(j + i, i)
                scat(j + i, i)
                nj = j + NB + i

                @pl.when(nj < NCH)
                def _():
                    l_issue(nj, i)
            return 0

        lax.fori_loop(0, NR, body, 0)
        for i in range(NCH - NR * NB):
            jt = NR * NB + i
            l_wait(jt, i)
            scat(jt, i)

        plsc.subcore_barrier()
        pltpu.sync_copy(acc_a.at[pl.ds(s * ZR, ZR)],
                        oa.at[c, pl.ds(s * ZR, ZR)])
        pltpu.sync_copy(acc_b.at[pl.ds(s * ZR, ZR)],
                        ob.at[c, pl.ds(s * ZR, ZR)])

    return k(vals_a, idx_a, vals_b, idx_b)


def _tc_node_emb(x, W, b):
    def body(x_r, W_r, b_r, o_r):
        o_r[...] = jnp.dot(x_r[...], W_r[...],
                           preferred_element_type=jnp.float32) + b_r[...]

    return pl.pallas_call(
        body,
        out_shape=jax.ShapeDtypeStruct((N, H), jnp.float32),
    )(x, W, b.reshape(1, H))


_BE = 2000


def _bd(W16):
    return jnp.kron(jnp.eye(8, dtype=jnp.float32), W16)


def _tc_edge_emb(eaV, W, b):
    BD = _bd(W)
    brow = jnp.tile(b, 8).reshape(1, 128)

    def body(ea_r, w_r, b_r, o_r):
        o_r[...] = jnp.dot(ea_r[...], w_r[...],
                           preferred_element_type=jnp.float32) + b_r[...]

    g = E8 // _BE
    return pl.pallas_call(
        body,
        grid=(g,),
        in_specs=[pl.BlockSpec((_BE, 128), lambda i: (i, 0)),
                  pl.BlockSpec((128, 128), lambda i: (0, 0)),
                  pl.BlockSpec((1, 128), lambda i: (0, 0))],
        out_specs=pl.BlockSpec((_BE, 128), lambda i: (i, 0)),
        out_shape=jax.ShapeDtypeStruct((E8, 128), jnp.float32),
    )(eaV, BD, brow)


def _tc_edge_alpha(hsV, hdV, eV, ai, ao):
    z8 = jnp.zeros((16, 8), jnp.float32)
    BDs = _bd(jnp.concatenate([ai["Ws"], ao["Ws"], z8], axis=1))
    BDd = _bd(jnp.concatenate([ai["Wd"], ao["Wd"], z8], axis=1))
    BDe = _bd(jnp.concatenate([ai["We"], ao["We"], z8], axis=1))
    brow = jnp.tile(
        jnp.concatenate([ai["bias"], ao["bias"],
                         jnp.zeros((8,), jnp.float32)]), 8).reshape(1, 128)

    def body(hs_r, hd_r, e_r, ws, wd, we, b_r, o_r):
        a = (jnp.dot(hs_r[...], ws[...], preferred_element_type=jnp.float32)
             + jnp.dot(hd_r[...], wd[...], preferred_element_type=jnp.float32)
             + jnp.dot(e_r[...], we[...], preferred_element_type=jnp.float32)
             + b_r[...])
        a = jnp.where(a > 0, a, 0.2 * a)
        o_r[...] = jnp.exp(a)

    g = E8 // _BE
    espec = pl.BlockSpec((_BE, 128), lambda i: (i, 0))
    wspec = pl.BlockSpec((128, 128), lambda i: (0, 0))
    return pl.pallas_call(
        body,
        grid=(g,),
        in_specs=[espec, espec, espec, wspec, wspec, wspec,
                  pl.BlockSpec((1, 128), lambda i: (0, 0))],
        out_specs=espec,
        out_shape=jax.ShapeDtypeStruct((E8, 128), jnp.float32),
    )(hsV, hdV, eV, BDs, BDd, BDe, brow)


def _tc_combine_S(Sa, Sb):
    def body(sa_r, sb_r, oa_r, ob_r):
        si = sa_r[0, :, 0:4] + sa_r[1, :, 0:4]
        so = sb_r[0, :, 4:8] + sb_r[1, :, 4:8]
        z4 = jnp.zeros((N, 4), jnp.float32)
        z8 = jnp.zeros((N, 8), jnp.float32)
        oa_r[...] = jnp.concatenate([1.0 / (si + 1e-16), z4, z8], axis=1)
        ob_r[...] = jnp.concatenate([z4, 1.0 / (so + 1e-16), z8], axis=1)

    return pl.pallas_call(
        body,
        out_shape=(jax.ShapeDtypeStruct((N, H), jnp.float32),
                   jax.ShapeDtypeStruct((N, H), jnp.float32)),
    )(Sa, Sb)


def _tc_edge_msg(PV, rgaV, rgbV, hsV, hdV, eV):
    MSUM = jnp.kron(jnp.eye(8, dtype=jnp.float32),
                    jnp.full((16, 16), 0.25, jnp.float32))

    def body(p_r, ra_r, rb_r, hs_r, hd_r, e_r, m_r, oi_r, oo_r):
        pv, ev, mv = p_r[...], e_r[...], m_r[...]
        amb_in = jnp.dot(pv * ra_r[...], mv,
                         preferred_element_type=jnp.float32,
                         precision=lax.Precision.HIGHEST)
        amb_out = jnp.dot(pv * rb_r[...], mv,
                          preferred_element_type=jnp.float32,
                          precision=lax.Precision.HIGHEST)
        oi_r[...] = jnp.maximum(hs_r[...] + ev * amb_in, 0.0)
        oo_r[...] = jnp.maximum(hd_r[...] + ev * amb_out, 0.0)

    g = E8 // _BE
    espec = pl.BlockSpec((_BE, 128), lambda i: (i, 0))
    return pl.pallas_call(
        body,
        grid=(g,),
        in_specs=[espec] * 6 + [pl.BlockSpec((128, 128), lambda i: (0, 0))],
        out_specs=(espec, espec),
        out_shape=(jax.ShapeDtypeStruct((E8, 128), jnp.float32),
                   jax.ShapeDtypeStruct((E8, 128), jnp.float32)),
    )(PV, rgaV, rgbV, hsV, hdV, eV, MSUM)


def _tc_node_update(h, Aa, Ab, lp):
    ci, co = lp["conv_in"], lp["conv_out"]
    eps2 = jnp.stack([ci["eps"], co["eps"]]).reshape(1, 2)

    def bn_(z, g, b):
        mu = jnp.mean(z, axis=0, keepdims=True)
        var = jnp.mean((z - mu) ** 2, axis=0, keepdims=True)
        return (z - mu) * lax.rsqrt(var + 1e-5) * g + b

    def conv_(h, aggr, eps, w1, b1, g1, bb1, w2, b2, g2, bb2):
        z = (1.0 + eps) * h + aggr
        z = jnp.dot(z, w1[...], preferred_element_type=jnp.float32) + b1[...]
        z = jnp.maximum(bn_(z, g1[...], bb1[...]), 0.0)
        z = jnp.dot(z, w2[...], preferred_element_type=jnp.float32) + b2[...]
        return jnp.maximum(bn_(z, g2[...], bb2[...]), 0.0)

    def body(h_r, aa_r, ab_r, eps_r,
             w1i, b1i, g1i, bb1i, w2i, b2i, g2i, bb2i,
             w1o, b1o, g1o, bb1o, w2o, b2o, g2o, bb2o,
             gn, bb, o_r):
        hv = h_r[...]
        aggr_in = aa_r[0] + aa_r[1]
        aggr_out = ab_r[0] + ab_r[1]
        h_in = conv_(hv, aggr_in, eps_r[0, 0],
                     w1i, b1i, g1i, bb1i, w2i, b2i, g2i, bb2i)
        h_out = conv_(hv, aggr_out, eps_r[0, 1],
                      w1o, b1o, g1o, bb1o, w2o, b2o, g2o, bb2o)
        m = bn_((h_in + h_out) * 0.5, gn[...], bb[...])
        o_r[...] = (hv + m) * 0.5

    def args(cp):
        return (cp["lin1"]["W"], cp["lin1"]["b"].reshape(1, -1),
                cp["bn1"]["g"].reshape(1, -1), cp["bn1"]["b"].reshape(1, -1),
                cp["lin2"]["W"], cp["lin2"]["b"].reshape(1, -1),
                cp["bn2"]["g"].reshape(1, -1), cp["bn2"]["b"].reshape(1, -1))

    return pl.pallas_call(
        body,
        out_shape=jax.ShapeDtypeStruct((N, H), jnp.float32),
    )(h, Aa, Ab, eps2, *args(ci), *args(co),
      lp["bn"]["g"].reshape(1, H), lp["bn"]["b"].reshape(1, H))


def _tc_final(h, W, b):
    def body(h_r, w_r, b_r, o_r):
        o_r[...] = jnp.dot(h_r[...], w_r[...],
                           preferred_element_type=jnp.float32) + b_r[...]

    return pl.pallas_call(
        body,
        out_shape=jax.ShapeDtypeStruct((N, 1), jnp.float32),
    )(h, W, b.reshape(1, 1))


def kernel(x, edge_index, edge_attr, params):
    src = edge_index[0]
    dst = edge_index[1]
    srcW = src.reshape(NW, NCH, C)
    dstW = dst.reshape(NW, NCH, C)

    def as_view(a):
        return a.reshape(E8, 128)

    def as_rows(v):
        return v.reshape(NW, NCH, C, H)

    h = _tc_node_emb(x, params["node_emb"]["W"], params["node_emb"]["b"])
    eV = _tc_edge_emb(edge_attr.reshape(E8, 128),
                      params["edge_emb"]["W"], params["edge_emb"]["b"])

    for lp in params["layers"]:
        hs, hd = _sc_gather2(h, srcW, h, dstW)
        hsV, hdV = as_view(hs), as_view(hd)
        PV = _tc_edge_alpha(hsV, hdV, eV, lp["att_in"], lp["att_out"])
        Sa, Sb = _sc_scatter2(as_rows(PV), dstW, as_rows(PV), srcW)
        rSa, rSb = _tc_combine_S(Sa, Sb)
        rga, rgb = _sc_gather2(rSa, dstW, rSb, srcW)
        mi, mo = _tc_edge_msg(PV, as_view(rga), as_view(rgb), hsV, hdV, eV)
        Aa, Ab = _sc_scatter2(as_rows(mi), dstW, as_rows(mo), srcW)
        h = _tc_node_update(h, Aa, Ab, lp)

    return _tc_final(h, params["mlp_node"]["W"], params["mlp_node"]["b"])


# speedup vs baseline: 19.0831x; 1.0098x over previous
import functools

import jax
import jax.numpy as jnp
from jax import lax
from jax.experimental import pallas as pl
from jax.experimental.pallas import tpu as pltpu
from jax.experimental.pallas import tpu_sc as plsc

N = 10000
E = 320000
F_IN = 128
H = 16
E8 = E * H // 128

NC = 2
NS = 16
NW = NC * NS
EW = E // NW
C = 80
NCH = EW // C
NB = 4
NR = (NCH - 1) // NB
ZR = N // NS

_mesh = lambda: plsc.VectorSubcoreMesh(core_axis_name="c", subcore_axis_name="s")
_sc_params = lambda: pltpu.CompilerParams(use_tc_tiling_on_sc=False)


def _sc_gather2(tab_a, idx_a, tab_b, idx_b):
    scratch = [pltpu.VMEM((NCH, C), jnp.int32), pltpu.VMEM((NCH, C), jnp.int32)]
    scratch += [pltpu.VMEM((C, H), jnp.float32)] * (2 * NB)
    scratch += [pltpu.SemaphoreType.DMA] * (4 * NB)

    @functools.partial(
        pl.kernel,
        out_type=(jax.ShapeDtypeStruct((E, H), jnp.float32),
                  jax.ShapeDtypeStruct((E, H), jnp.float32)),
        mesh=_mesh(),
        compiler_params=_sc_params(),
        scratch_types=scratch,
    )
    def k(ta, ia, tb, ib, oa, ob, iva, ivb, *bufsem):
        bum = bufsem[:2 * NB]
        sems = bufsem[2 * NB:]
        ba = bum[:NB]
        bb = bum[NB:]
        ga = sems[:NB]
        gb = sems[NB:2 * NB]
        ta_s = sems[2 * NB:3 * NB]
        tb_s = sems[3 * NB:]
        w = lax.axis_index("s") * NC + lax.axis_index("c")
        base = w * EW
        pltpu.sync_copy(ia.at[w], iva)
        pltpu.sync_copy(ib.at[w], ivb)

        def g_issue(j, i):
            pltpu.async_copy(ta.at[iva.at[j]], ba[i], ga[i])
            pltpu.async_copy(tb.at[ivb.at[j]], bb[i], gb[i])

        def g_wait(j, i):
            pltpu.make_async_copy(ta.at[iva.at[j]], ba[i], ga[i]).wait()
            pltpu.make_async_copy(tb.at[ivb.at[j]], bb[i], gb[i]).wait()

        def st_issue(j, i):
            pltpu.async_copy(ba[i], oa.at[pl.ds(base + j * C, C)], ta_s[i])
            pltpu.async_copy(bb[i], ob.at[pl.ds(base + j * C, C)], tb_s[i])

        def st_wait(j, i):
            pltpu.make_async_copy(ba[i], oa.at[pl.ds(base + j * C, C)],
                                  ta_s[i]).wait()
            pltpu.make_async_copy(bb[i], ob.at[pl.ds(base + j * C, C)],
                                  tb_s[i]).wait()

        for i in range(NB):
            g_issue(i, i)

        def body(t, _):
            j = NB * t
            for i in range(NB):
                g_wait(j + i, i)
                st_issue(j + i, i)
            for i in range(NB):
                st_wait(j + i, i)
                nj = j + NB + i

                @pl.when(nj < NCH)
                def _():
                    g_issue(nj, i)
            return 0

        lax.fori_loop(0, NR, body, 0)
        for i in range(NCH - NR * NB):
            jt = NR * NB + i
            g_wait(jt, i)
            st_issue(jt, i)
        for i in range(NCH - NR * NB):
            jt = NR * NB + i
            st_wait(jt, i)

    return k(tab_a, idx_a, tab_b, idx_b)


def _sc_scatter2(vals_a, idx_a, vals_b, idx_b, shared_vals=False):
    scratch = [pltpu.VMEM((NCH, C), jnp.int32), pltpu.VMEM((NCH, C), jnp.int32)]
    scratch += [pltpu.VMEM((C, H), jnp.float32)] * (2 * NB)
    scratch += [pltpu.VMEM((ZR, H), jnp.float32),
                pltpu.VMEM_SHARED((N, H), jnp.float32),
                pltpu.VMEM_SHARED((N, H), jnp.float32)]
    scratch += [pltpu.SemaphoreType.DMA] * (2 * NB)

    @functools.partial(
        pl.kernel,
        out_type=(jax.ShapeDtypeStruct((NC, N, H), jnp.float32),
                  jax.ShapeDtypeStruct((NC, N, H), jnp.float32)),
        mesh=_mesh(),
        compiler_params=_sc_params(),
        scratch_types=scratch,
    )
    def k(va, ia, vb, ib, oa, ob, iva, ivb, *rest):
        ba = rest[:NB]
        bb = rest[NB:2 * NB]
        zbuf, acc_a, acc_b = rest[2 * NB:2 * NB + 3]
        sems = rest[2 * NB + 3:]
        la = sems[:NB]
        lb = sems[NB:]
        c = lax.axis_index("c")
        s = lax.axis_index("s")
        w = s * NC + c

        def zrow(i, _):
            zbuf[i, :] = jnp.zeros((H,), jnp.float32)
            return 0
        lax.fori_loop(0, ZR, zrow, 0)
        pltpu.sync_copy(zbuf, acc_a.at[pl.ds(s * ZR, ZR)])
        pltpu.sync_copy(zbuf, acc_b.at[pl.ds(s * ZR, ZR)])

        pltpu.sync_copy(ia.at[w], iva)
        pltpu.sync_copy(ib.at[w], ivb)
        plsc.subcore_barrier()

        def l_issue(j, i):
            pltpu.async_copy(va.at[w, j], ba[i], la[i])
            if not shared_vals:
                pltpu.async_copy(vb.at[w, j], bb[i], lb[i])

        def l_wait(j, i):
            pltpu.make_async_copy(va.at[w, j], ba[i], la[i]).wait()
            if not shared_vals:
                pltpu.make_async_copy(vb.at[w, j], bb[i], lb[i]).wait()

        def scat(j, i):
            pltpu.sync_copy(ba[i], acc_a.at[iva.at[j]], add=True)
            pltpu.sync_copy(ba[i] if shared_vals else bb[i],
                            acc_b.at[ivb.at[j]], add=True)

        for i in range(NB):
            l_issue(i, i)

        def body(t, _):
            j = NB * t
            for i in range(NB):
                l_wait(j + i, i)
                scat(j + i, i)
                nj = j + NB + i

                @pl.when(nj < NCH)
                def _():
                    l_issue(nj, i)
            return 0

        lax.fori_loop(0, NR, body, 0)
        for i in range(NCH - NR * NB):
            jt = NR * NB + i
            l_wait(jt, i)
            scat(jt, i)

        plsc.subcore_barrier()
        pltpu.sync_copy(acc_a.at[pl.ds(s * ZR, ZR)],
                        oa.at[c, pl.ds(s * ZR, ZR)])
        pltpu.sync_copy(acc_b.at[pl.ds(s * ZR, ZR)],
                        ob.at[c, pl.ds(s * ZR, ZR)])

    return k(vals_a, idx_a, vals_b, idx_b)


def _tc_node_emb(x, W, b):
    def body(x_r, W_r, b_r, o_r):
        o_r[...] = jnp.dot(x_r[...], W_r[...],
                           preferred_element_type=jnp.float32) + b_r[...]

    return pl.pallas_call(
        body,
        out_shape=jax.ShapeDtypeStruct((N, H), jnp.float32),
    )(x, W, b.reshape(1, H))


_BE = 2000


def _bd(W16):
    return jnp.kron(jnp.eye(8, dtype=jnp.float32), W16)


def _tc_edge_emb(eaV, W, b):
    BD = _bd(W)
    brow = jnp.tile(b, 8).reshape(1, 128)

    def body(ea_r, w_r, b_r, o_r):
        o_r[...] = jnp.dot(ea_r[...], w_r[...],
                           preferred_element_type=jnp.float32) + b_r[...]

    g = E8 // _BE
    return pl.pallas_call(
        body,
        grid=(g,),
        in_specs=[pl.BlockSpec((_BE, 128), lambda i: (i, 0)),
                  pl.BlockSpec((128, 128), lambda i: (0, 0)),
                  pl.BlockSpec((1, 128), lambda i: (0, 0))],
        out_specs=pl.BlockSpec((_BE, 128), lambda i: (i, 0)),
        out_shape=jax.ShapeDtypeStruct((E8, 128), jnp.float32),
    )(eaV, BD, brow)


def _tc_edge_alpha(hsV, hdV, eV, ai, ao):
    z8 = jnp.zeros((16, 8), jnp.float32)
    BDs = _bd(jnp.concatenate([ai["Ws"], ao["Ws"], z8], axis=1))
    BDd = _bd(jnp.concatenate([ai["Wd"], ao["Wd"], z8], axis=1))
    BDe = _bd(jnp.concatenate([ai["We"], ao["We"], z8], axis=1))
    brow = jnp.tile(
        jnp.concatenate([ai["bias"], ao["bias"],
                         jnp.zeros((8,), jnp.float32)]), 8).reshape(1, 128)

    def body(hs_r, hd_r, e_r, ws, wd, we, b_r, o_r):
        a = (jnp.dot(hs_r[...], ws[...], preferred_element_type=jnp.float32)
             + jnp.dot(hd_r[...], wd[...], preferred_element_type=jnp.float32)
             + jnp.dot(e_r[...], we[...], preferred_element_type=jnp.float32)
             + b_r[...])
        a = jnp.where(a > 0, a, 0.2 * a)
        o_r[...] = jnp.exp(a)

    g = E8 // _BE
    espec = pl.BlockSpec((_BE, 128), lambda i: (i, 0))
    wspec = pl.BlockSpec((128, 128), lambda i: (0, 0))
    return pl.pallas_call(
        body,
        grid=(g,),
        in_specs=[espec, espec, espec, wspec, wspec, wspec,
                  pl.BlockSpec((1, 128), lambda i: (0, 0))],
        out_specs=espec,
        out_shape=jax.ShapeDtypeStruct((E8, 128), jnp.float32),
    )(hsV, hdV, eV, BDs, BDd, BDe, brow)


def _tc_combine_S(Sa, Sb):
    def body(sa_r, sb_r, oa_r, ob_r):
        si = sa_r[0, :, 0:4] + sa_r[1, :, 0:4]
        so = sb_r[0, :, 4:8] + sb_r[1, :, 4:8]
        z4 = jnp.zeros((N, 4), jnp.float32)
        z8 = jnp.zeros((N, 8), jnp.float32)
        oa_r[...] = jnp.concatenate([1.0 / (si + 1e-16), z4, z8], axis=1)
        ob_r[...] = jnp.concatenate([z4, 1.0 / (so + 1e-16), z8], axis=1)

    return pl.pallas_call(
        body,
        out_shape=(jax.ShapeDtypeStruct((N, H), jnp.float32),
                   jax.ShapeDtypeStruct((N, H), jnp.float32)),
    )(Sa, Sb)


def _tc_edge_msg(PV, rgaV, rgbV, hsV, hdV, eV):
    MSUM = jnp.kron(jnp.eye(8, dtype=jnp.float32),
                    jnp.full((16, 16), 0.25, jnp.float32))

    def body(p_r, ra_r, rb_r, hs_r, hd_r, e_r, m_r, oi_r, oo_r):
        pv, ev, mv = p_r[...], e_r[...], m_r[...]
        amb_in = jnp.dot(pv * ra_r[...], mv,
                         preferred_element_type=jnp.float32,
                         precision=lax.Precision.HIGHEST)
        amb_out = jnp.dot(pv * rb_r[...], mv,
                          preferred_element_type=jnp.float32,
                          precision=lax.Precision.HIGHEST)
        oi_r[...] = jnp.maximum(hs_r[...] + ev * amb_in, 0.0)
        oo_r[...] = jnp.maximum(hd_r[...] + ev * amb_out, 0.0)

    g = E8 // _BE
    espec = pl.BlockSpec((_BE, 128), lambda i: (i, 0))
    return pl.pallas_call(
        body,
        grid=(g,),
        in_specs=[espec] * 6 + [pl.BlockSpec((128, 128), lambda i: (0, 0))],
        out_specs=(espec, espec),
        out_shape=(jax.ShapeDtypeStruct((E8, 128), jnp.float32),
                   jax.ShapeDtypeStruct((E8, 128), jnp.float32)),
    )(PV, rgaV, rgbV, hsV, hdV, eV, MSUM)


def _tc_node_update(h, Aa, Ab, lp):
    ci, co = lp["conv_in"], lp["conv_out"]
    eps2 = jnp.stack([ci["eps"], co["eps"]]).reshape(1, 2)

    def bn_(z, g, b):
        mu = jnp.mean(z, axis=0, keepdims=True)
        var = jnp.mean((z - mu) ** 2, axis=0, keepdims=True)
        return (z - mu) * lax.rsqrt(var + 1e-5) * g + b

    def conv_(h, aggr, eps, w1, b1, g1, bb1, w2, b2, g2, bb2):
        z = (1.0 + eps) * h + aggr
        z = jnp.dot(z, w1[...], preferred_element_type=jnp.float32) + b1[...]
        z = jnp.maximum(bn_(z, g1[...], bb1[...]), 0.0)
        z = jnp.dot(z, w2[...], preferred_element_type=jnp.float32) + b2[...]
        return jnp.maximum(bn_(z, g2[...], bb2[...]), 0.0)

    def body(h_r, aa_r, ab_r, eps_r,
             w1i, b1i, g1i, bb1i, w2i, b2i, g2i, bb2i,
             w1o, b1o, g1o, bb1o, w2o, b2o, g2o, bb2o,
             gn, bb, o_r):
        hv = h_r[...]
        aggr_in = aa_r[0] + aa_r[1]
        aggr_out = ab_r[0] + ab_r[1]
        h_in = conv_(hv, aggr_in, eps_r[0, 0],
                     w1i, b1i, g1i, bb1i, w2i, b2i, g2i, bb2i)
        h_out = conv_(hv, aggr_out, eps_r[0, 1],
                      w1o, b1o, g1o, bb1o, w2o, b2o, g2o, bb2o)
        m = bn_((h_in + h_out) * 0.5, gn[...], bb[...])
        o_r[...] = (hv + m) * 0.5

    def args(cp):
        return (cp["lin1"]["W"], cp["lin1"]["b"].reshape(1, -1),
                cp["bn1"]["g"].reshape(1, -1), cp["bn1"]["b"].reshape(1, -1),
                cp["lin2"]["W"], cp["lin2"]["b"].reshape(1, -1),
                cp["bn2"]["g"].reshape(1, -1), cp["bn2"]["b"].reshape(1, -1))

    return pl.pallas_call(
        body,
        out_shape=jax.ShapeDtypeStruct((N, H), jnp.float32),
    )(h, Aa, Ab, eps2, *args(ci), *args(co),
      lp["bn"]["g"].reshape(1, H), lp["bn"]["b"].reshape(1, H))


def _tc_final(h, W, b):
    def body(h_r, w_r, b_r, o_r):
        o_r[...] = jnp.dot(h_r[...], w_r[...],
                           preferred_element_type=jnp.float32) + b_r[...]

    return pl.pallas_call(
        body,
        out_shape=jax.ShapeDtypeStruct((N, 1), jnp.float32),
    )(h, W, b.reshape(1, 1))


def kernel(x, edge_index, edge_attr, params):
    src = edge_index[0]
    dst = edge_index[1]
    srcW = src.reshape(NW, NCH, C)
    dstW = dst.reshape(NW, NCH, C)

    def as_view(a):
        return a.reshape(E8, 128)

    def as_rows(v):
        return v.reshape(NW, NCH, C, H)

    h = _tc_node_emb(x, params["node_emb"]["W"], params["node_emb"]["b"])
    eV = _tc_edge_emb(edge_attr.reshape(E8, 128),
                      params["edge_emb"]["W"], params["edge_emb"]["b"])

    for lp in params["layers"]:
        hs, hd = _sc_gather2(h, srcW, h, dstW)
        hsV, hdV = as_view(hs), as_view(hd)
        PV = _tc_edge_alpha(hsV, hdV, eV, lp["att_in"], lp["att_out"])
        Sa, Sb = _sc_scatter2(as_rows(PV), dstW, as_rows(PV), srcW,
                              shared_vals=True)
        rSa, rSb = _tc_combine_S(Sa, Sb)
        rga, rgb = _sc_gather2(rSa, dstW, rSb, srcW)
        mi, mo = _tc_edge_msg(PV, as_view(rga), as_view(rgb), hsV, hdV, eV)
        Aa, Ab = _sc_scatter2(as_rows(mi), dstW, as_rows(mo), srcW)
        h = _tc_node_update(h, Aa, Ab, lp)

    return _tc_final(h, params["mlp_node"]["W"], params["mlp_node"]["b"])
